# baseline (speedup 1.0000x reference)
import functools

import jax
import jax.numpy as jnp
from jax import lax
from jax.experimental import pallas as pl
from jax.experimental.pallas import tpu as pltpu

VMEM = pltpu.MemorySpace.VMEM


def _prep_kernel(h_ref, w_ref, al_ref, ar_ref, bd_ref, rc_ref, f0_ref, f1_ref,
                 *, num_heads, head_dim):
    n = h_ref.shape[0]
    feat = jnp.dot(h_ref[...].astype(jnp.bfloat16),
                   w_ref[...].astype(jnp.bfloat16),
                   preferred_element_type=jnp.float32)
    feat_bf = feat.astype(jnp.bfloat16)
    ones = jnp.ones((n, 1), jnp.bfloat16)
    zer = jnp.zeros((n, head_dim - 1), jnp.bfloat16)

    bd_rows, cls, faugs = [], [], []
    for hd in range(num_heads):
        lo = hd * head_dim
        fh = feat[:, lo:lo + head_dim]
        al = al_ref[hd:hd + 1, :]
        el = lax.dot_general(al, fh, (((1,), (1,)), ((), ())),
                             preferred_element_type=jnp.float32)
        cl = jnp.max(el, axis=1, keepdims=True)
        cls.append(cl)
        bd_rows.append(jnp.exp(el - cl))
        bd_rows.append(jnp.exp(0.2 * (el - cl)))
        faugs.append(jnp.concatenate([feat_bf[:, lo:lo + head_dim],
                                      ones, zer], axis=1))

    pad_r = jnp.zeros((8 - 2 * num_heads, n), jnp.float32)
    bd_ref[...] = jnp.concatenate(bd_rows + [pad_r],
                                  axis=0).astype(jnp.bfloat16)

    art = jnp.transpose(ar_ref[...].astype(jnp.float32))
    zc = jnp.zeros((head_dim, 1), jnp.float32)
    gcols = [jnp.concatenate([zc] * hd + [art[:, hd:hd + 1]]
                             + [zc] * (num_heads - 1 - hd), axis=1)
             for hd in range(num_heads)]
    gmat = jnp.concatenate(gcols, axis=0) * (-0.8)
    clrow = jnp.concatenate(cls, axis=1) * (-0.8)
    u2 = jnp.dot(feat, gmat,
                 preferred_element_type=jnp.float32) + clrow
    pad_c = jnp.zeros((n, 8 - num_heads), jnp.float32)
    rc_ref[...] = jnp.exp(jnp.concatenate([u2, pad_c],
                                          axis=1)).astype(jnp.bfloat16)
    f0_ref[...] = faugs[0]
    f1_ref[...] = faugs[1]


def _prep(h, w, al, ar, *, num_heads, head_dim):
    m, n, fin = h.shape
    d = num_heads * head_dim
    body = functools.partial(_prep_kernel, num_heads=num_heads,
                             head_dim=head_dim)
    return pl.pallas_call(
        body,
        out_shape=(
            jax.ShapeDtypeStruct((m, 8, n), jnp.bfloat16),
            jax.ShapeDtypeStruct((m, n, 8), jnp.bfloat16),
            jax.ShapeDtypeStruct((m, n, 2 * head_dim), jnp.bfloat16),
            jax.ShapeDtypeStruct((m, n, 2 * head_dim), jnp.bfloat16),
        ),
        grid=(m,),
        in_specs=[
            pl.BlockSpec((None, n, fin), lambda i: (i, 0, 0)),
            pl.BlockSpec((None, fin, d), lambda i: (i, 0, 0)),
            pl.BlockSpec((None, num_heads, head_dim), lambda i: (i, 0, 0)),
            pl.BlockSpec((None, num_heads, head_dim), lambda i: (i, 0, 0)),
        ],
        out_specs=(
            pl.BlockSpec((None, 8, n), lambda i: (i, 0, 0)),
            pl.BlockSpec((None, n, 8), lambda i: (i, 0, 0)),
            pl.BlockSpec((None, n, 2 * head_dim), lambda i: (i, 0, 0)),
            pl.BlockSpec((None, n, 2 * head_dim), lambda i: (i, 0, 0)),
        ),
        compiler_params=pltpu.CompilerParams(
            dimension_semantics=("arbitrary",)),
    )(h, w, al, ar)


def _gat_kernel(adj_ref, bd_ref, rc_ref, f0_ref, f1_ref, b_ref,
                sw1_ref, sb1_ref, sw2_ref, o_ref,
                *, num_heads, head_dim, tile_m, npg, nc):
    d = num_heads * head_dim
    n = bd_ref.shape[1]
    kc = n // nc
    bias = b_ref[...]
    rs = [rc_ref[:, hd:hd + 1] for hd in range(num_heads)]

    mms = [None] * num_heads
    for ks in range(nc):
        mask_bf = adj_ref[:, ks * kc:(ks + 1) * kc].astype(jnp.bfloat16)
        for hd in range(num_heads):
            b_src = bd_ref[2 * hd:2 * hd + 1, ks * kc:(ks + 1) * kc]
            d_src = bd_ref[2 * hd + 1:2 * hd + 2, ks * kc:(ks + 1) * kc]
            p = jnp.maximum(b_src, rs[hd] * d_src) * mask_bf
            fa_ref = f0_ref if hd == 0 else f1_ref
            contrib = jnp.dot(p, fa_ref[ks * kc:(ks + 1) * kc, :],
                              preferred_element_type=jnp.float32)
            mms[hd] = contrib if ks == 0 else mms[hd] + contrib

    num = jnp.concatenate([mm[:, :head_dim] for mm in mms], axis=1)
    den = jnp.concatenate(
        [jnp.broadcast_to(jnp.maximum(mm[:, head_dim:head_dim + 1], 1e-30),
                          (tile_m, head_dim)) for mm in mms], axis=1)
    o = num / den + bias
    o = jnp.where(o > 0, o, jnp.exp(jnp.minimum(o, 0.0)) - 1.0)

    gpt = tile_m // npg
    g = jnp.sum(o.reshape(gpt, npg, d), axis=1)

    t = jnp.tanh(jnp.dot(o, sw1_ref[...],
                         preferred_element_type=jnp.float32) + sb1_ref[...])
    sc = jnp.dot(t, sw2_ref[...], preferred_element_type=jnp.float32)
    spart = jnp.full((1, d), jnp.sum(sc), jnp.float32)

    rows_out = o_ref.shape[1]
    pieces = [g, spart]
    if rows_out > gpt + 1:
        pieces.append(jnp.zeros((rows_out - gpt - 1, d), jnp.float32))
    o_ref[...] = jnp.concatenate(pieces, axis=0)[None]


def _gat_all(adj, bd, rc, f0, f1, bias, sw1, sb1, sw2,
             *, num_heads, head_dim, tile_m, npg, nc):
    m, n, _ = adj.shape
    d = num_heads * head_dim
    tiles = n // tile_m

    flops = int(m * (num_heads * 2 * n * n * head_dim + 2 * n * d * 128))
    transcendentals = int(m * n * 130)
    bytes_accessed = int(m * n * n * adj.dtype.itemsize)

    gpt = tile_m // npg
    rows_out = ((gpt + 1 + 7) // 8) * 8
    body = functools.partial(_gat_kernel, num_heads=num_heads,
                             head_dim=head_dim, tile_m=tile_m, npg=npg, nc=nc)
    return pl.pallas_call(
        body,
        out_shape=jax.ShapeDtypeStruct((m, tiles, rows_out, d), jnp.float32),
        grid=(m, tiles),
        in_specs=[
            pl.BlockSpec((None, tile_m, n), lambda i, t: (i, t, 0)),
            pl.BlockSpec((None, 8, n), lambda i, t: (i, 0, 0)),
            pl.BlockSpec((None, tile_m, 8), lambda i, t: (i, t, 0)),
            pl.BlockSpec((None, n, 2 * head_dim), lambda i, t: (i, 0, 0)),
            pl.BlockSpec((None, n, 2 * head_dim), lambda i, t: (i, 0, 0)),
            pl.BlockSpec((None, 1, d), lambda i, t: (i, 0, 0)),
            pl.BlockSpec(sw1.shape, lambda i, t: (0, 0)),
            pl.BlockSpec(sb1.shape, lambda i, t: (0, 0)),
            pl.BlockSpec(sw2.shape, lambda i, t: (0, 0)),
        ],
        out_specs=pl.BlockSpec((None, 1, rows_out, d),
                               lambda i, t: (i, t, 0, 0)),
        compiler_params=pltpu.CompilerParams(
            dimension_semantics=("arbitrary", "arbitrary"),
            vmem_limit_bytes=64 * 1024 * 1024),
        cost_estimate=pl.CostEstimate(flops=flops,
                                      transcendentals=transcendentals,
                                      bytes_accessed=bytes_accessed),
    )(adj, bd, rc, f0, f1, bias, sw1, sb1, sw2)


def _head_kernel(ga_ref, pw_ref, pb_ref, mw1_ref, mb1_ref, mw2_ref, mb2_ref,
                 mw3_ref, mb3_ref, o_ref, *, n_nodes, npg, tile_m):
    ga = ga_ref[...]
    mcount, tiles, _, d = ga.shape
    gpt = tile_m // npg

    sc = ga[:, :, gpt:gpt + 1, 0:1]
    scores = jnp.sum(sc, axis=1, keepdims=True) * (1.0 / n_nodes)
    mx = jnp.max(scores, axis=0, keepdims=True)
    e = jnp.exp(scores - mx)
    beta = e / jnp.sum(e, axis=0, keepdims=True)

    gsum = jnp.sum(ga[:, :, 0:gpt, :] * beta, axis=0)
    gm = gsum.reshape(tiles * gpt, d)

    g = (jnp.dot(gm, pw_ref[...], preferred_element_type=jnp.float32)
         + npg * pb_ref[...])
    x = jnp.maximum(jnp.dot(g, mw1_ref[...],
                            preferred_element_type=jnp.float32)
                    + mb1_ref[...], 0.0)
    x = jnp.maximum(jnp.dot(x, mw2_ref[...],
                            preferred_element_type=jnp.float32)
                    + mb2_ref[...], 0.0)
    logits = jnp.dot(x, mw3_ref[...],
                     preferred_element_type=jnp.float32) + mb3_ref[...]
    mmax = jnp.max(logits, axis=1, keepdims=True)
    p = jnp.exp(logits - mmax)
    o_ref[...] = p / jnp.sum(p, axis=1, keepdims=True)


def _head(ga, pred_w, pred_b, mlp, *, batch, npg, tile_m):
    n_nodes = batch * npg
    body = functools.partial(_head_kernel, n_nodes=n_nodes, npg=npg,
                             tile_m=tile_m)
    vspec = pl.BlockSpec(memory_space=VMEM)
    return pl.pallas_call(
        body,
        out_shape=jax.ShapeDtypeStruct((batch, 2), jnp.float32),
        in_specs=[vspec] * 9,
        out_specs=vspec,
    )(ga, pred_w, pred_b, mlp["w1"], mlp["b1"], mlp["w2"], mlp["b2"],
      mlp["w3"], mlp["b3"])


def _han(gat_w, gat_al, gat_ar, gat_bias, sem_w1, sem_b1, sem_w2,
         pred_w, pred_b, mlp, adj, h, *, num_heads, head_dim, batch, npg,
         tile_m, nc):
    bd, rc, f0, f1 = _prep(h, gat_w, gat_al, gat_ar,
                           num_heads=num_heads, head_dim=head_dim)
    ga = _gat_all(adj, bd, rc, f0, f1, gat_bias, sem_w1, sem_b1, sem_w2,
                  num_heads=num_heads, head_dim=head_dim, tile_m=tile_m,
                  npg=npg, nc=nc)
    return _head(ga, pred_b=pred_b, pred_w=pred_w, mlp=mlp, batch=batch,
                 npg=npg, tile_m=tile_m)


def kernel(gat_w, gat_al, gat_ar, gat_bias, sem_w1, sem_b1, sem_w2,
           pred_w, pred_b, mlp_w1, mlp_b1, mlp_w2, mlp_b2, mlp_w3, mlp_b3,
           adj, h):
    mlp = {"w1": mlp_w1, "b1": mlp_b1, "w2": mlp_w2, "b2": mlp_b2,
           "w3": mlp_w3, "b3": mlp_b3}
    return _han(gat_w, gat_al, gat_ar, gat_bias, sem_w1, sem_b1, sem_w2,
                pred_w, pred_b, mlp, adj, h,
                num_heads=2, head_dim=8, batch=64, npg=64, tile_m=1024, nc=4)


# speedup vs baseline: 1.0156x; 1.0156x over previous
import functools

import jax
import jax.numpy as jnp
from jax import lax
from jax.experimental import pallas as pl
from jax.experimental.pallas import tpu as pltpu

VMEM = pltpu.MemorySpace.VMEM


def _prep_kernel(h_ref, w_ref, al_ref, ar_ref, bd_ref, rc_ref, f0_ref, f1_ref,
                 *, num_heads, head_dim):
    n = h_ref.shape[0]
    feat = jnp.dot(h_ref[...].astype(jnp.bfloat16),
                   w_ref[...].astype(jnp.bfloat16),
                   preferred_element_type=jnp.float32)
    feat_bf = feat.astype(jnp.bfloat16)
    ones = jnp.ones((n, 1), jnp.bfloat16)
    zer = jnp.zeros((n, head_dim - 1), jnp.bfloat16)

    bd_rows, cls, faugs = [], [], []
    for hd in range(num_heads):
        lo = hd * head_dim
        fh = feat[:, lo:lo + head_dim]
        al = al_ref[hd:hd + 1, :]
        el = lax.dot_general(al, fh, (((1,), (1,)), ((), ())),
                             preferred_element_type=jnp.float32)
        cl = jnp.max(el, axis=1, keepdims=True)
        cls.append(cl)
        bd_rows.append(jnp.exp(el - cl))
        bd_rows.append(jnp.exp(0.2 * (el - cl)))
        faugs.append(jnp.concatenate([feat_bf[:, lo:lo + head_dim],
                                      ones, zer], axis=1))

    pad_r = jnp.zeros((8 - 2 * num_heads, n), jnp.float32)
    bd_ref[...] = jnp.concatenate(bd_rows + [pad_r],
                                  axis=0).astype(jnp.bfloat16)

    art = jnp.transpose(ar_ref[...].astype(jnp.float32))
    zc = jnp.zeros((head_dim, 1), jnp.float32)
    gcols = [jnp.concatenate([zc] * hd + [art[:, hd:hd + 1]]
                             + [zc] * (num_heads - 1 - hd), axis=1)
             for hd in range(num_heads)]
    gmat = jnp.concatenate(gcols, axis=0) * (-0.8)
    clrow = jnp.concatenate(cls, axis=1) * (-0.8)
    u2 = jnp.dot(feat, gmat,
                 preferred_element_type=jnp.float32) + clrow
    pad_c = jnp.zeros((n, 8 - num_heads), jnp.float32)
    rc_ref[...] = jnp.exp(jnp.concatenate([u2, pad_c],
                                          axis=1)).astype(jnp.bfloat16)
    f0_ref[...] = faugs[0]
    f1_ref[...] = faugs[1]


def _prep(h, w, al, ar, *, num_heads, head_dim):
    m, n, fin = h.shape
    d = num_heads * head_dim
    body = functools.partial(_prep_kernel, num_heads=num_heads,
                             head_dim=head_dim)
    return pl.pallas_call(
        body,
        out_shape=(
            jax.ShapeDtypeStruct((m, 8, n), jnp.bfloat16),
            jax.ShapeDtypeStruct((m, n, 8), jnp.bfloat16),
            jax.ShapeDtypeStruct((m, n, 2 * head_dim), jnp.bfloat16),
            jax.ShapeDtypeStruct((m, n, 2 * head_dim), jnp.bfloat16),
        ),
        grid=(m,),
        in_specs=[
            pl.BlockSpec((None, n, fin), lambda i: (i, 0, 0)),
            pl.BlockSpec((None, fin, d), lambda i: (i, 0, 0)),
            pl.BlockSpec((None, num_heads, head_dim), lambda i: (i, 0, 0)),
            pl.BlockSpec((None, num_heads, head_dim), lambda i: (i, 0, 0)),
        ],
        out_specs=(
            pl.BlockSpec((None, 8, n), lambda i: (i, 0, 0)),
            pl.BlockSpec((None, n, 8), lambda i: (i, 0, 0)),
            pl.BlockSpec((None, n, 2 * head_dim), lambda i: (i, 0, 0)),
            pl.BlockSpec((None, n, 2 * head_dim), lambda i: (i, 0, 0)),
        ),
        compiler_params=pltpu.CompilerParams(
            dimension_semantics=("arbitrary",)),
    )(h, w, al, ar)


def _gat_kernel(adj_ref, bd_ref, rc_ref, f0_ref, f1_ref, b_ref,
                sw1_ref, sb1_ref, sw2_ref, o_ref,
                *, num_heads, head_dim, tile_m, npg, nc):
    d = num_heads * head_dim
    n = bd_ref.shape[1]
    kc = n // nc
    bias = b_ref[...]
    rs = [rc_ref[:, hd:hd + 1] for hd in range(num_heads)]

    mms = [None] * num_heads
    for ks in range(nc):
        mask_bf = adj_ref[:, ks * kc:(ks + 1) * kc].astype(jnp.bfloat16)
        for hd in range(num_heads):
            b_src = bd_ref[2 * hd:2 * hd + 1, ks * kc:(ks + 1) * kc]
            d_src = bd_ref[2 * hd + 1:2 * hd + 2, ks * kc:(ks + 1) * kc]
            p = jnp.maximum(b_src, rs[hd] * d_src) * mask_bf
            fa_ref = f0_ref if hd == 0 else f1_ref
            contrib = jnp.dot(p, fa_ref[ks * kc:(ks + 1) * kc, :],
                              preferred_element_type=jnp.float32)
            mms[hd] = contrib if ks == 0 else mms[hd] + contrib

    num = jnp.concatenate([mm[:, :head_dim] for mm in mms], axis=1)
    den = jnp.concatenate(
        [jnp.broadcast_to(jnp.maximum(mm[:, head_dim:head_dim + 1], 1e-30),
                          (tile_m, head_dim)) for mm in mms], axis=1)
    o = num / den + bias
    o = jnp.where(o > 0, o, jnp.exp(jnp.minimum(o, 0.0)) - 1.0)

    gpt = tile_m // npg
    g = jnp.sum(o.reshape(gpt, npg, d), axis=1)

    t = jnp.tanh(jnp.dot(o, sw1_ref[...],
                         preferred_element_type=jnp.float32) + sb1_ref[...])
    sc = jnp.dot(t, sw2_ref[...], preferred_element_type=jnp.float32)
    spart = jnp.full((1, d), jnp.sum(sc), jnp.float32)

    rows_out = o_ref.shape[1]
    pieces = [g, spart]
    if rows_out > gpt + 1:
        pieces.append(jnp.zeros((rows_out - gpt - 1, d), jnp.float32))
    o_ref[...] = jnp.concatenate(pieces, axis=0)[None]


def _gat_all(adj, bd, rc, f0, f1, bias, sw1, sb1, sw2,
             *, num_heads, head_dim, tile_m, npg, nc):
    m, n, _ = adj.shape
    d = num_heads * head_dim
    tiles = n // tile_m

    flops = int(m * (num_heads * 2 * n * n * head_dim + 2 * n * d * 128))
    transcendentals = int(m * n * 130)
    bytes_accessed = int(m * n * n * adj.dtype.itemsize)

    gpt = tile_m // npg
    rows_out = ((gpt + 1 + 7) // 8) * 8
    body = functools.partial(_gat_kernel, num_heads=num_heads,
                             head_dim=head_dim, tile_m=tile_m, npg=npg, nc=nc)
    return pl.pallas_call(
        body,
        out_shape=jax.ShapeDtypeStruct((m, tiles, rows_out, d), jnp.float32),
        grid=(m, tiles),
        in_specs=[
            pl.BlockSpec((None, tile_m, n), lambda i, t: (i, t, 0)),
            pl.BlockSpec((None, 8, n), lambda i, t: (i, 0, 0)),
            pl.BlockSpec((None, tile_m, 8), lambda i, t: (i, t, 0)),
            pl.BlockSpec((None, n, 2 * head_dim), lambda i, t: (i, 0, 0)),
            pl.BlockSpec((None, n, 2 * head_dim), lambda i, t: (i, 0, 0)),
            pl.BlockSpec((None, 1, d), lambda i, t: (i, 0, 0)),
            pl.BlockSpec(sw1.shape, lambda i, t: (0, 0)),
            pl.BlockSpec(sb1.shape, lambda i, t: (0, 0)),
            pl.BlockSpec(sw2.shape, lambda i, t: (0, 0)),
        ],
        out_specs=pl.BlockSpec((None, 1, rows_out, d),
                               lambda i, t: (i, t, 0, 0)),
        compiler_params=pltpu.CompilerParams(
            dimension_semantics=("arbitrary", "arbitrary"),
            vmem_limit_bytes=64 * 1024 * 1024),
        cost_estimate=pl.CostEstimate(flops=flops,
                                      transcendentals=transcendentals,
                                      bytes_accessed=bytes_accessed),
    )(adj, bd, rc, f0, f1, bias, sw1, sb1, sw2)


def _head_kernel(ga_ref, pw_ref, pb_ref, mw1_ref, mb1_ref, mw2_ref, mb2_ref,
                 mw3_ref, mb3_ref, o_ref, *, n_nodes, npg, tile_m):
    ga = ga_ref[...]
    mcount, tiles, _, d = ga.shape
    gpt = tile_m // npg

    sc = ga[:, :, gpt:gpt + 1, 0:1]
    scores = jnp.sum(sc, axis=1, keepdims=True) * (1.0 / n_nodes)
    mx = jnp.max(scores, axis=0, keepdims=True)
    e = jnp.exp(scores - mx)
    beta = e / jnp.sum(e, axis=0, keepdims=True)

    gsum = jnp.sum(ga[:, :, 0:gpt, :] * beta, axis=0)
    gm = gsum.reshape(tiles * gpt, d)

    g = (jnp.dot(gm, pw_ref[...], preferred_element_type=jnp.float32)
         + npg * pb_ref[...])
    x = jnp.maximum(jnp.dot(g, mw1_ref[...],
                            preferred_element_type=jnp.float32)
                    + mb1_ref[...], 0.0)
    x = jnp.maximum(jnp.dot(x, mw2_ref[...],
                            preferred_element_type=jnp.float32)
                    + mb2_ref[...], 0.0)
    logits = jnp.dot(x, mw3_ref[...],
                     preferred_element_type=jnp.float32) + mb3_ref[...]
    mmax = jnp.max(logits, axis=1, keepdims=True)
    p = jnp.exp(logits - mmax)
    o_ref[...] = p / jnp.sum(p, axis=1, keepdims=True)


def _head(ga, pred_w, pred_b, mlp, *, batch, npg, tile_m):
    n_nodes = batch * npg
    body = functools.partial(_head_kernel, n_nodes=n_nodes, npg=npg,
                             tile_m=tile_m)
    vspec = pl.BlockSpec(memory_space=VMEM)
    return pl.pallas_call(
        body,
        out_shape=jax.ShapeDtypeStruct((batch, 2), jnp.float32),
        in_specs=[vspec] * 9,
        out_specs=vspec,
    )(ga, pred_w, pred_b, mlp["w1"], mlp["b1"], mlp["w2"], mlp["b2"],
      mlp["w3"], mlp["b3"])


def _han(gat_w, gat_al, gat_ar, gat_bias, sem_w1, sem_b1, sem_w2,
         pred_w, pred_b, mlp, adj, h, *, num_heads, head_dim, batch, npg,
         tile_m, nc):
    bd, rc, f0, f1 = _prep(h, gat_w, gat_al, gat_ar,
                           num_heads=num_heads, head_dim=head_dim)
    ga = _gat_all(adj, bd, rc, f0, f1, gat_bias, sem_w1, sem_b1, sem_w2,
                  num_heads=num_heads, head_dim=head_dim, tile_m=tile_m,
                  npg=npg, nc=nc)
    return _head(ga, pred_b=pred_b, pred_w=pred_w, mlp=mlp, batch=batch,
                 npg=npg, tile_m=tile_m)


def kernel(gat_w, gat_al, gat_ar, gat_bias, sem_w1, sem_b1, sem_w2,
           pred_w, pred_b, mlp_w1, mlp_b1, mlp_w2, mlp_b2, mlp_w3, mlp_b3,
           adj, h):
    mlp = {"w1": mlp_w1, "b1": mlp_b1, "w2": mlp_w2, "b2": mlp_b2,
           "w3": mlp_w3, "b3": mlp_b3}
    return _han(gat_w, gat_al, gat_ar, gat_bias, sem_w1, sem_b1, sem_w2,
                pred_w, pred_b, mlp, adj, h,
                num_heads=2, head_dim=8, batch=64, npg=64, tile_m=2048, nc=4)
